# Initial kernel scaffold; baseline (speedup 1.0000x reference)
import jax
import jax.numpy as jnp
from jax.experimental import pallas as pl


def kernel(x, edge_index, edge_attr, batch, li_ni_indices, k_neighbors, u_nb, v_nb, num_nodes, params):
    raise NotImplementedError("write your pallas kernel here")


# speedup vs baseline: 4.9550x; 4.9550x over previous
import functools

import jax
import jax.numpy as jnp
from jax import lax
from jax.experimental import pallas as pl
from jax.experimental.pallas import tpu as pltpu
from jax.experimental.pallas import tpu_sc as plsc

GAT_EPS = 1e-05
L = 16
NT = 16
CH = 80


def _mm_body(x_ref, w_ref, b_ref, wa_ref, o_ref, oa_ref):
    xh = jnp.dot(x_ref[...], w_ref[...], preferred_element_type=jnp.float32)
    xh = xh + b_ref[...]
    o_ref[...] = xh
    oa_ref[...] = jnp.dot(xh, wa_ref[...], preferred_element_type=jnp.float32)


def _tc_matmul(x, w, b, wa):
    n, k = x.shape
    m = w.shape[1]
    ma = wa.shape[1]
    blk = 2000
    grid = n // blk
    return pl.pallas_call(
        _mm_body,
        grid=(grid,),
        in_specs=[
            pl.BlockSpec((blk, k), lambda i: (i, 0)),
            pl.BlockSpec((k, m), lambda i: (0, 0)),
            pl.BlockSpec((1, m), lambda i: (0, 0)),
            pl.BlockSpec((m, ma), lambda i: (0, 0)),
        ],
        out_specs=[
            pl.BlockSpec((blk, m), lambda i: (i, 0)),
            pl.BlockSpec((blk, ma), lambda i: (i, 0)),
        ],
        out_shape=[
            jax.ShapeDtypeStruct((n, m), jnp.float32),
            jax.ShapeDtypeStruct((n, ma), jnp.float32),
        ],
    )(x, w, b, wa)


def _bnrelu_body(e_ref, g_ref, b_ref, o_ref):
    n = o_ref.shape[0]
    for h in range(2):
        xh = e_ref[pl.ds(h * n, n), :]
        mu = jnp.mean(xh, axis=0, keepdims=True)
        var = jnp.mean((xh - mu) ** 2, axis=0, keepdims=True)
        gh = g_ref[:, pl.ds(h * 128, 128)]
        bh = b_ref[:, pl.ds(h * 128, 128)]
        y = gh * (xh - mu) / jnp.sqrt(var + GAT_EPS) + bh
        o_ref[:, pl.ds(h * 128, 128)] = jnp.maximum(y, 0.0)


def _tc_bnrelu(e, g, b):
    n = e.shape[0] // 2
    return pl.pallas_call(
        _bnrelu_body,
        out_shape=jax.ShapeDtypeStruct((n, 256), jnp.float32),
    )(e, g.reshape(1, 256), b.reshape(1, 256))


def _head_body(xf_ref, ili_ref, ini_ref, gs_ref, bs_ref, w1, b1, w2, b2, w3,
               b3, w4, b4, wp, bp, o_ref, kh_ref):
    nb = o_ref.shape[0]
    for j in range(nb):
        a = xf_ref[pl.ds(ili_ref[j], 1), :]
        c = xf_ref[pl.ds(ini_ref[j], 1), :]
        kh_ref[pl.ds(j, 1), :] = (a + c) * 0.5

    def bn(x, i, d):
        mu = jnp.mean(x, axis=0, keepdims=True)
        var = jnp.mean((x - mu) ** 2, axis=0, keepdims=True)
        g = gs_ref[pl.ds(i, 1), pl.ds(0, d)]
        b = bs_ref[pl.ds(i, 1), pl.ds(0, d)]
        return g * (x - mu) / jnp.sqrt(var + GAT_EPS) + b

    def lin(x, w_ref, b_ref):
        return (
            jnp.dot(x, w_ref[...], preferred_element_type=jnp.float32)
            + b_ref[...]
        )

    kh = kh_ref[...]
    x1 = lin(jnp.maximum(bn(kh, 0, 128), 0.0), w1, b1)
    x2 = lin(jnp.maximum(bn(x1, 1, 256), 0.0), w2, b2) + x1
    x3 = lin(jnp.maximum(bn(x2, 2, 256), 0.0), w3, b3) + x2
    x4 = lin(jnp.maximum(bn(x3, 3, 256), 0.0), w4, b4) + x3
    o_ref[...] = lin(x4, wp, bp)


def _tc_head(x_f, adj_li, adj_ni, p):
    nb = adj_li.shape[0]
    gs = jnp.zeros((4, 256), jnp.float32)
    bs = jnp.zeros((4, 256), jnp.float32)
    gs = gs.at[0, :128].set(p["bnh1"]["g"]).at[1].set(p["bnh2"]["g"])
    gs = gs.at[2].set(p["bnh3"]["g"]).at[3].set(p["bnh4"]["g"])
    bs = bs.at[0, :128].set(p["bnh1"]["b"]).at[1].set(p["bnh2"]["b"])
    bs = bs.at[2].set(p["bnh3"]["b"]).at[3].set(p["bnh4"]["b"])
    args = [x_f, adj_li, adj_ni, gs, bs]
    in_specs = [
        pl.BlockSpec(memory_space=pltpu.VMEM),
        pl.BlockSpec(memory_space=pltpu.SMEM),
        pl.BlockSpec(memory_space=pltpu.SMEM),
        pl.BlockSpec(memory_space=pltpu.VMEM),
        pl.BlockSpec(memory_space=pltpu.VMEM),
    ]
    for nm in ("h1", "h2", "h3", "h4", "pred"):
        args.append(p[nm]["W"].T)
        args.append(p[nm]["b"].reshape(1, -1))
        in_specs.append(pl.BlockSpec(memory_space=pltpu.VMEM))
        in_specs.append(pl.BlockSpec(memory_space=pltpu.VMEM))
    return pl.pallas_call(
        _head_body,
        in_specs=in_specs,
        out_shape=jax.ShapeDtypeStruct((nb, 1), jnp.float32),
        scratch_shapes=[pltpu.VMEM((nb, 128), jnp.float32)],
    )(*args)


def _sc_edge_body(n, e, xh_hbm, ad_hbm, as_hbm, src_hbm, dst_hbm, ea_hbm,
                  zero_hbm, out_hbm, ad_v, as_v, src_v, dst_v, ea_v, gix_v,
                  w_v, rows_v, gsem, acc):
    c = lax.axis_index("c")
    s = lax.axis_index("s")
    rpt = (n // NT) // 8 * 8
    tail = n - NT * rpt
    ept = e // NT
    nch = ept // CH

    pltpu.sync_copy(zero_hbm.at[pl.ds(s * rpt, rpt)], acc.at[pl.ds(s * rpt, rpt)])
    if tail:
        @pl.when(s == NT - 1)
        def _():
            pltpu.sync_copy(zero_hbm.at[pl.ds(NT * rpt, tail)],
                            acc.at[pl.ds(NT * rpt, tail)])
    pltpu.sync_copy(ad_hbm.at[pl.ds(c * n, n)], ad_v)
    pltpu.sync_copy(as_hbm.at[pl.ds(c * n, n)], as_v)
    plsc.subcore_barrier()

    base0 = s * ept

    def chunk(k, carry):
        base = base0 + k * CH
        pltpu.sync_copy(src_hbm.at[pl.ds(base, CH)], src_v)
        pltpu.sync_copy(dst_hbm.at[pl.ds(base, CH)], dst_v)
        pltpu.sync_copy(ea_hbm.at[pl.ds(base, CH)], ea_v)
        for i in range(CH // L):
            sl = pl.ds(i * L, L)
            sv = src_v[sl]
            gix_v[sl] = sv + c * n
            z = (plsc.load_gather(ad_v, [dst_v[sl]])
                 + plsc.load_gather(as_v, [sv]))
            z = jnp.maximum(z, 0.01 * z)
            w_v[sl] = z * ea_v[sl]
        pltpu.async_copy(xh_hbm.at[gix_v], rows_v, gsem).wait()

        def scale(gi, cc):
            w16 = w_v[pl.ds(gi * L, L)]
            for j in range(L):
                wj = w16[j]
                ei = gi * L + j
                for g in range(8):
                    slg = pl.ds(g * L, L)
                    rows_v[ei, slg] = rows_v[ei, slg] * wj
            return cc

        lax.fori_loop(0, CH // L, scale, 0)
        pltpu.sync_copy(rows_v, acc.at[dst_v], add=True)
        return carry

    lax.fori_loop(0, nch, chunk, 0)
    plsc.subcore_barrier()
    pltpu.sync_copy(acc.at[pl.ds(s * rpt, rpt)],
                    out_hbm.at[pl.ds(c * n + s * rpt, rpt)])
    if tail:
        @pl.when(s == NT - 1)
        def _():
            pltpu.sync_copy(acc.at[pl.ds(NT * rpt, tail)],
                            out_hbm.at[pl.ds(c * n + NT * rpt, tail)])


def _sc_edge(xh2, ad2, as2, src, dst, ea_inv, zeros):
    n2 = xh2.shape[0]
    n = n2 // 2
    e = src.shape[0]
    assert n % NT == 0 and e % (NT * CH) == 0
    mesh = plsc.VectorSubcoreMesh(core_axis_name="c", subcore_axis_name="s")
    kfn = pl.kernel(
        functools.partial(_sc_edge_body, n, e),
        out_type=jax.ShapeDtypeStruct((n2, 128), jnp.float32),
        mesh=mesh,
        compiler_params=pltpu.CompilerParams(needs_layout_passes=False),
        scratch_types=[
            pltpu.VMEM((n,), jnp.float32),
            pltpu.VMEM((n,), jnp.float32),
            pltpu.VMEM((CH,), jnp.int32),
            pltpu.VMEM((CH,), jnp.int32),
            pltpu.VMEM((CH,), jnp.float32),
            pltpu.VMEM((CH,), jnp.int32),
            pltpu.VMEM((CH,), jnp.float32),
            pltpu.VMEM((CH, 128), jnp.float32),
            pltpu.SemaphoreType.DMA,
            pltpu.VMEM_SHARED((n, 128), jnp.float32),
        ],
    )
    return kfn(xh2, ad2, as2, src, dst, ea_inv, zeros)


def _aug_weights(p):
    w = p["W"]
    b = p["b"]
    attd = p["att"][:, 0, :128]
    atts = p["att"][:, 0, 128:]
    wcat = jnp.concatenate([w[0].T, w[1].T], axis=1)
    bcat = jnp.concatenate([b[0], b[1]]).reshape(1, 256)
    z = jnp.zeros((128,), jnp.float32)
    wa = jnp.stack([
        jnp.concatenate([attd[0], z]),
        jnp.concatenate([atts[0], z]),
        jnp.concatenate([z, attd[1]]),
        jnp.concatenate([z, atts[1]]),
    ], axis=1)
    return wcat, bcat, wa


def kernel(x, edge_index, edge_attr, batch, li_ni_indices, k_neighbors,
           u_nb, v_nb, num_nodes, params):
    n = x.shape[0]
    src = edge_index[0].astype(jnp.int32)
    dst = edge_index[1].astype(jnp.int32)
    ea_inv = 1.0 / (edge_attr + 1e-07) ** 2
    zeros = jnp.zeros((n, 128), jnp.float32)

    def conv(xin, p):
        wcat, bcat, wa = _aug_weights(p)
        y, a = _tc_matmul(xin, wcat, bcat, wa)
        xh2 = jnp.concatenate([y[:, :128], y[:, 128:256]], axis=0)
        ad2 = jnp.concatenate([a[:, 0], a[:, 2]])
        as2 = jnp.concatenate([a[:, 1], a[:, 3]])
        return _sc_edge(xh2, ad2, as2, src, dst, ea_inv, zeros)

    e1 = conv(x, params["conv1"])
    x1 = _tc_bnrelu(e1, params["bn1"]["g"], params["bn1"]["b"])
    e2 = conv(x1, params["conv2"])
    x2 = _tc_bnrelu(e2, params["bn2"]["g"], params["bn2"]["b"])
    e3 = conv(x2, params["conv3"]) + conv(x1, params["conv3"])
    x3 = _tc_bnrelu(e3, params["bn3"]["g"], params["bn3"]["b"])
    e4 = conv(x3, params["conv4"])
    x4 = _tc_bnrelu(e4, params["bn4"]["g"], params["bn4"]["b"])
    ef = conv(x4, params["conv_f"])
    x_f = 0.5 * (ef[:n] + ef[n:])

    nb = li_ni_indices.shape[0]
    nppg = num_nodes / (jnp.max(batch) + 1)
    start = (jnp.arange(nb) * nppg).astype(jnp.float32)
    adj_li = (li_ni_indices[:, 0].astype(jnp.float32) + start).astype(jnp.int32)
    adj_ni = (li_ni_indices[:, 1].astype(jnp.float32) + start).astype(jnp.int32)
    return _tc_head(x_f, adj_li, adj_ni, params)


# speedup vs baseline: 10.6607x; 2.1515x over previous
import functools

import jax
import jax.numpy as jnp
from jax import lax
from jax.experimental import pallas as pl
from jax.experimental.pallas import tpu as pltpu
from jax.experimental.pallas import tpu_sc as plsc

GAT_EPS = 1e-05
L = 16
NT = 16
CH = 80


def _mm_body(x_ref, w_ref, b_ref, wa_ref, o_ref, oa_ref):
    xh = jnp.dot(x_ref[...], w_ref[...], preferred_element_type=jnp.float32)
    xh = xh + b_ref[...]
    o_ref[...] = xh
    oa_ref[...] = jnp.dot(xh, wa_ref[...], preferred_element_type=jnp.float32)


def _tc_matmul(x, w, b, wa):
    n, k = x.shape
    m = w.shape[1]
    ma = wa.shape[1]
    blk = 2000
    grid = n // blk
    return pl.pallas_call(
        _mm_body,
        grid=(grid,),
        in_specs=[
            pl.BlockSpec((blk, k), lambda i: (i, 0)),
            pl.BlockSpec((k, m), lambda i: (0, 0)),
            pl.BlockSpec((1, m), lambda i: (0, 0)),
            pl.BlockSpec((m, ma), lambda i: (0, 0)),
        ],
        out_specs=[
            pl.BlockSpec((blk, m), lambda i: (i, 0)),
            pl.BlockSpec((blk, ma), lambda i: (i, 0)),
        ],
        out_shape=[
            jax.ShapeDtypeStruct((n, m), jnp.float32),
            jax.ShapeDtypeStruct((n, ma), jnp.float32),
        ],
    )(x, w, b, wa)


def _bnrelu_body(e_ref, g_ref, b_ref, o_ref):
    n = o_ref.shape[0]
    for h in range(2):
        xh = e_ref[pl.ds(h * n, n), :]
        mu = jnp.mean(xh, axis=0, keepdims=True)
        var = jnp.mean((xh - mu) ** 2, axis=0, keepdims=True)
        gh = g_ref[:, pl.ds(h * 128, 128)]
        bh = b_ref[:, pl.ds(h * 128, 128)]
        y = gh * (xh - mu) / jnp.sqrt(var + GAT_EPS) + bh
        o_ref[:, pl.ds(h * 128, 128)] = jnp.maximum(y, 0.0)


def _tc_bnrelu(e, g, b):
    n = e.shape[0] // 2
    return pl.pallas_call(
        _bnrelu_body,
        out_shape=jax.ShapeDtypeStruct((n, 256), jnp.float32),
    )(e, g.reshape(1, 256), b.reshape(1, 256))


def _head_body(xf_ref, ili_ref, ini_ref, gs_ref, bs_ref, w1, b1, w2, b2, w3,
               b3, w4, b4, wp, bp, o_ref, kh_ref):
    nb = o_ref.shape[0]
    for j in range(nb):
        a = xf_ref[pl.ds(ili_ref[j], 1), :]
        c = xf_ref[pl.ds(ini_ref[j], 1), :]
        kh_ref[pl.ds(j, 1), :] = (a + c) * 0.5

    def bn(x, i, d):
        mu = jnp.mean(x, axis=0, keepdims=True)
        var = jnp.mean((x - mu) ** 2, axis=0, keepdims=True)
        g = gs_ref[pl.ds(i, 1), pl.ds(0, d)]
        b = bs_ref[pl.ds(i, 1), pl.ds(0, d)]
        return g * (x - mu) / jnp.sqrt(var + GAT_EPS) + b

    def lin(x, w_ref, b_ref):
        return (
            jnp.dot(x, w_ref[...], preferred_element_type=jnp.float32)
            + b_ref[...]
        )

    kh = kh_ref[...]
    x1 = lin(jnp.maximum(bn(kh, 0, 128), 0.0), w1, b1)
    x2 = lin(jnp.maximum(bn(x1, 1, 256), 0.0), w2, b2) + x1
    x3 = lin(jnp.maximum(bn(x2, 2, 256), 0.0), w3, b3) + x2
    x4 = lin(jnp.maximum(bn(x3, 3, 256), 0.0), w4, b4) + x3
    o_ref[...] = lin(x4, wp, bp)


def _tc_head(x_f, adj_li, adj_ni, p):
    nb = adj_li.shape[0]
    gs = jnp.zeros((4, 256), jnp.float32)
    bs = jnp.zeros((4, 256), jnp.float32)
    gs = gs.at[0, :128].set(p["bnh1"]["g"]).at[1].set(p["bnh2"]["g"])
    gs = gs.at[2].set(p["bnh3"]["g"]).at[3].set(p["bnh4"]["g"])
    bs = bs.at[0, :128].set(p["bnh1"]["b"]).at[1].set(p["bnh2"]["b"])
    bs = bs.at[2].set(p["bnh3"]["b"]).at[3].set(p["bnh4"]["b"])
    args = [x_f, adj_li, adj_ni, gs, bs]
    in_specs = [
        pl.BlockSpec(memory_space=pltpu.VMEM),
        pl.BlockSpec(memory_space=pltpu.SMEM),
        pl.BlockSpec(memory_space=pltpu.SMEM),
        pl.BlockSpec(memory_space=pltpu.VMEM),
        pl.BlockSpec(memory_space=pltpu.VMEM),
    ]
    for nm in ("h1", "h2", "h3", "h4", "pred"):
        args.append(p[nm]["W"].T)
        args.append(p[nm]["b"].reshape(1, -1))
        in_specs.append(pl.BlockSpec(memory_space=pltpu.VMEM))
        in_specs.append(pl.BlockSpec(memory_space=pltpu.VMEM))
    return pl.pallas_call(
        _head_body,
        in_specs=in_specs,
        out_shape=jax.ShapeDtypeStruct((nb, 1), jnp.float32),
        scratch_shapes=[pltpu.VMEM((nb, 128), jnp.float32)],
    )(*args)


def _sc_edge_body(n, e, xh_hbm, ad_hbm, as_hbm, src_hbm, dst_hbm, ea_hbm,
                  zero_hbm, out_hbm, ad_v, as_v, src0, src1, dst0, dst1,
                  ea0, ea1, gix0, gix1, rows0, rows1, w_v, ssem0, ssem1,
                  dsem0, dsem1, easem0, easem1, gsem0, gsem1, acc):
    c = lax.axis_index("c")
    s = lax.axis_index("s")
    rpt = (n // NT) // 8 * 8
    tail = n - NT * rpt
    ept = e // NT
    nch = ept // CH
    nch2 = nch // 2

    pltpu.sync_copy(zero_hbm.at[pl.ds(s * rpt, rpt)], acc.at[pl.ds(s * rpt, rpt)])
    if tail:
        @pl.when(s == NT - 1)
        def _():
            pltpu.sync_copy(zero_hbm.at[pl.ds(NT * rpt, tail)],
                            acc.at[pl.ds(NT * rpt, tail)])
    pltpu.sync_copy(ad_hbm.at[pl.ds(c * n, n)], ad_v)
    pltpu.sync_copy(as_hbm.at[pl.ds(c * n, n)], as_v)
    plsc.subcore_barrier()

    base0 = s * ept

    def issue_ie(k, src_v, dst_v, ea_v, ssem, dsem, easem):
        base = base0 + k * CH
        pltpu.async_copy(src_hbm.at[pl.ds(base, CH)], src_v, ssem)
        pltpu.async_copy(dst_hbm.at[pl.ds(base, CH)], dst_v, dsem)
        pltpu.async_copy(ea_hbm.at[pl.ds(base, CH)], ea_v, easem)

    def start_gather(k, src_v, gix_v, rows_v, ssem, gsem):
        base = base0 + k * CH
        pltpu.make_async_copy(src_hbm.at[pl.ds(base, CH)], src_v,
                              ssem).wait()
        for i in range(CH // L):
            sl = pl.ds(i * L, L)
            gix_v[sl] = src_v[sl] + c * n
        pltpu.async_copy(xh_hbm.at[gix_v], rows_v, gsem)

    def finish(k, src_v, dst_v, ea_v, gix_v, rows_v, dsem, easem, gsem):
        base = base0 + k * CH
        pltpu.make_async_copy(xh_hbm.at[gix_v], rows_v, gsem).wait()
        pltpu.make_async_copy(dst_hbm.at[pl.ds(base, CH)], dst_v,
                              dsem).wait()
        pltpu.make_async_copy(ea_hbm.at[pl.ds(base, CH)], ea_v,
                              easem).wait()
        for i in range(CH // L):
            sl = pl.ds(i * L, L)
            z = (plsc.load_gather(ad_v, [dst_v[sl]])
                 + plsc.load_gather(as_v, [src_v[sl]]))
            z = jnp.maximum(z, 0.01 * z)
            w_v[sl] = z * ea_v[sl]

        def scale(gi, cc):
            w16 = w_v[pl.ds(gi * L, L)]
            for j in range(L):
                wj = w16[j]
                ei = gi * L + j
                for g in range(8):
                    slg = pl.ds(g * L, L)
                    rows_v[ei, slg] = rows_v[ei, slg] * wj
            return cc

        lax.fori_loop(0, CH // L, scale, 0)
        pltpu.sync_copy(rows_v, acc.at[dst_v], add=True)

    issue_ie(0, src0, dst0, ea0, ssem0, dsem0, easem0)
    issue_ie(1, src1, dst1, ea1, ssem1, dsem1, easem1)
    start_gather(0, src0, gix0, rows0, ssem0, gsem0)

    def pair(i, carry):
        k0 = 2 * i
        start_gather(k0 + 1, src1, gix1, rows1, ssem1, gsem1)
        finish(k0, src0, dst0, ea0, gix0, rows0, dsem0, easem0, gsem0)

        @pl.when(i < nch2 - 1)
        def _():
            issue_ie(k0 + 2, src0, dst0, ea0, ssem0, dsem0, easem0)
            start_gather(k0 + 2, src0, gix0, rows0, ssem0, gsem0)

        finish(k0 + 1, src1, dst1, ea1, gix1, rows1, dsem1, easem1, gsem1)

        @pl.when(i < nch2 - 1)
        def _():
            issue_ie(k0 + 3, src1, dst1, ea1, ssem1, dsem1, easem1)

        return carry

    lax.fori_loop(0, nch2, pair, 0)
    plsc.subcore_barrier()
    pltpu.sync_copy(acc.at[pl.ds(s * rpt, rpt)],
                    out_hbm.at[pl.ds(c * n + s * rpt, rpt)])
    if tail:
        @pl.when(s == NT - 1)
        def _():
            pltpu.sync_copy(acc.at[pl.ds(NT * rpt, tail)],
                            out_hbm.at[pl.ds(c * n + NT * rpt, tail)])


def _sc_edge(xh2, ad2, as2, src, dst, ea_inv, zeros):
    n2 = xh2.shape[0]
    n = n2 // 2
    e = ea_inv.shape[0]
    assert n % NT == 0 and e % (NT * CH * 2) == 0
    mesh = plsc.VectorSubcoreMesh(core_axis_name="c", subcore_axis_name="s")
    kfn = pl.kernel(
        functools.partial(_sc_edge_body, n, e),
        out_type=jax.ShapeDtypeStruct((n2, 128), jnp.float32),
        mesh=mesh,
        compiler_params=pltpu.CompilerParams(needs_layout_passes=False),
        scratch_types=[
            pltpu.VMEM((n,), jnp.float32),
            pltpu.VMEM((n,), jnp.float32),
            pltpu.VMEM((CH,), jnp.int32),
            pltpu.VMEM((CH,), jnp.int32),
            pltpu.VMEM((CH,), jnp.int32),
            pltpu.VMEM((CH,), jnp.int32),
            pltpu.VMEM((CH,), jnp.float32),
            pltpu.VMEM((CH,), jnp.float32),
            pltpu.VMEM((CH,), jnp.int32),
            pltpu.VMEM((CH,), jnp.int32),
            pltpu.VMEM((CH, 128), jnp.float32),
            pltpu.VMEM((CH, 128), jnp.float32),
            pltpu.VMEM((CH,), jnp.float32),
            pltpu.SemaphoreType.DMA,
            pltpu.SemaphoreType.DMA,
            pltpu.SemaphoreType.DMA,
            pltpu.SemaphoreType.DMA,
            pltpu.SemaphoreType.DMA,
            pltpu.SemaphoreType.DMA,
            pltpu.SemaphoreType.DMA,
            pltpu.SemaphoreType.DMA,
            pltpu.VMEM_SHARED((n, 128), jnp.float32),
        ],
    )
    return kfn(xh2, ad2, as2, src, dst, ea_inv, zeros)


def _aug_weights(p):
    w = p["W"]
    b = p["b"]
    attd = p["att"][:, 0, :128]
    atts = p["att"][:, 0, 128:]
    wcat = jnp.concatenate([w[0].T, w[1].T], axis=1)
    bcat = jnp.concatenate([b[0], b[1]]).reshape(1, 256)
    z = jnp.zeros((128,), jnp.float32)
    wa = jnp.stack([
        jnp.concatenate([attd[0], z]),
        jnp.concatenate([atts[0], z]),
        jnp.concatenate([z, attd[1]]),
        jnp.concatenate([z, atts[1]]),
    ], axis=1)
    return wcat, bcat, wa


def kernel(x, edge_index, edge_attr, batch, li_ni_indices, k_neighbors,
           u_nb, v_nb, num_nodes, params):
    n = x.shape[0]
    src = edge_index[0].astype(jnp.int32)
    dst = edge_index[1].astype(jnp.int32)
    ea_inv = 1.0 / (edge_attr + 1e-07) ** 2
    zeros = jnp.zeros((n, 128), jnp.float32)

    def conv(xin, p):
        wcat, bcat, wa = _aug_weights(p)
        y, a = _tc_matmul(xin, wcat, bcat, wa)
        xh2 = jnp.concatenate([y[:, :128], y[:, 128:256]], axis=0)
        ad2 = jnp.concatenate([a[:, 0], a[:, 2]])
        as2 = jnp.concatenate([a[:, 1], a[:, 3]])
        return _sc_edge(xh2, ad2, as2, src, dst, ea_inv, zeros)

    e1 = conv(x, params["conv1"])
    x1 = _tc_bnrelu(e1, params["bn1"]["g"], params["bn1"]["b"])
    e2 = conv(x1, params["conv2"])
    x2 = _tc_bnrelu(e2, params["bn2"]["g"], params["bn2"]["b"])
    e3 = conv(x2, params["conv3"]) + conv(x1, params["conv3"])
    x3 = _tc_bnrelu(e3, params["bn3"]["g"], params["bn3"]["b"])
    e4 = conv(x3, params["conv4"])
    x4 = _tc_bnrelu(e4, params["bn4"]["g"], params["bn4"]["b"])
    ef = conv(x4, params["conv_f"])
    x_f = 0.5 * (ef[:n] + ef[n:])

    nb = li_ni_indices.shape[0]
    nppg = num_nodes / (jnp.max(batch) + 1)
    start = (jnp.arange(nb) * nppg).astype(jnp.float32)
    adj_li = (li_ni_indices[:, 0].astype(jnp.float32) + start).astype(jnp.int32)
    adj_ni = (li_ni_indices[:, 1].astype(jnp.float32) + start).astype(jnp.int32)
    return _tc_head(x_f, adj_li, adj_ni, params)
